# baseline (speedup 1.0000x reference)
import functools

import jax
import jax.numpy as jnp
from jax import lax
from jax.experimental import pallas as pl
from jax.experimental.pallas import tpu as pltpu
from jax.experimental.pallas import tpu_sc as plsc

VOCAB = 1000000
EMB = 64
PAD_W = 128
SEQ = 200
BATCH = 4096
NUM_CORES = 2
NUM_SUBCORES = 16
NUM_WORKERS = NUM_CORES * NUM_SUBCORES
SBLK = BATCH // NUM_WORKERS
NG = 4
NT = 4


def _make_gather():
    mesh = plsc.VectorSubcoreMesh(core_axis_name="c", subcore_axis_name="s")

    @functools.partial(
        pl.kernel,
        mesh=mesh,
        out_type=jax.ShapeDtypeStruct((SEQ, EMB, BATCH), jnp.float32),
        scratch_types=[
            pltpu.VMEM((SEQ, SBLK), jnp.int32),
            [pltpu.VMEM((SBLK, PAD_W), jnp.float32) for _ in range(NG)],
            [pltpu.VMEM((EMB, SBLK), jnp.float32) for _ in range(NT)],
            [pltpu.VMEM((SBLK,), jnp.int32) for _ in range(NG)],
            [pltpu.SemaphoreType.DMA for _ in range(NG)],
            [pltpu.SemaphoreType.DMA for _ in range(NT)],
        ],
        compiler_params=pltpu.CompilerParams(
            use_tc_tiling_on_sc=True, needs_layout_passes=False),
    )
    def k(idx_hbm, table_hbm, out_hbm, idx_v, gbuf, tbuf, kidx, gsem, osem):
        wid = lax.axis_index("s") * NUM_CORES + lax.axis_index("c")
        s0 = wid * SBLK

        pltpu.sync_copy(idx_hbm.at[:, pl.ds(s0, SBLK)], idx_v)

        lanes = lax.iota(jnp.int32, 16)
        rot = [jax.lax.rem(lanes + jnp.int32(kk), jnp.int32(16))
               for kk in range(16)]

        def gather_desc(g):
            return pltpu.make_async_copy(
                table_hbm.at[kidx[g]], gbuf[g], gsem[g])

        def put_desc(t, b):
            return pltpu.make_async_copy(
                tbuf[b], out_hbm.at[t, :, pl.ds(s0, SBLK)], osem[b])

        def prep_gather(t, g):
            for m in range(SBLK // 16):
                iv = idx_v[t, pl.ds(m * 16, 16)]
                kidx[g][pl.ds(m * 16, 16)] = jax.lax.shift_right_logical(iv, 1)
            gather_desc(g).start()

        def transpose_block(t, g, b):
            def mbody(m, carry):
                for half in range(2):
                    off = m * 32 + half * 16
                    iv = idx_v[t, pl.ds(off, 16)]
                    pbase = jax.lax.shift_left(
                        jax.lax.bitwise_and(iv, jnp.int32(1)), 6)
                    rows_m = off + lanes
                    for db in range(EMB // 16):
                        d0 = db * 16
                        vs = [plsc.load_gather(
                            gbuf[g], [rows_m, pbase + jnp.int32(d0) + rot[kk]])
                            for kk in range(16)]
                        for kk in range(16):
                            plsc.store_scatter(
                                tbuf[b],
                                [jnp.int32(d0) + rot[kk], off + lanes],
                                vs[kk])
                return carry

            lax.fori_loop(0, SBLK // 32, mbody, 0)

        def step(t, gu, first=False, last=False):
            g = gu % NG
            b = gu % NT
            if not last:
                prep_gather(t + 2, (gu + 2) % NG)
            if not first:
                put_desc(t - NT, b).wait()
            gather_desc(g).wait()
            transpose_block(t, g, b)
            put_desc(t, b).start()

        prep_gather(0, 0)
        prep_gather(1, 1)
        for u in range(4):
            step(u, u, first=True)

        def body(r, carry):
            t = 4 + r * 4
            for u in range(4):
                step(t + u, u)
            return carry

        lax.fori_loop(0, (SEQ - 8) // 4, body, 0)

        for t in range(SEQ - 4, SEQ):
            step(t, t % 4, last=(t + 2 >= SEQ))
        for t in range(SEQ - 4, SEQ):
            put_desc(t, t % NT).wait()

    return k


def kernel(indices, weight):
    idx_t = jnp.transpose(indices)
    table = weight.reshape(VOCAB // 2, PAD_W)
    out_t = _make_gather()(idx_t, table)
    return jnp.transpose(out_t, (2, 0, 1))


# speedup vs baseline: 1.0250x; 1.0250x over previous
import functools

import jax
import jax.numpy as jnp
from jax import lax
from jax.experimental import pallas as pl
from jax.experimental.pallas import tpu as pltpu
from jax.experimental.pallas import tpu_sc as plsc

VOCAB = 1000000
EMB = 64
PAD_W = 128
SEQ = 200
BATCH = 4096
NUM_CORES = 2
NUM_SUBCORES = 16
NUM_WORKERS = NUM_CORES * NUM_SUBCORES
SBLK = BATCH // NUM_WORKERS
NG = 4
NT = 4


def _make_gather():
    mesh = plsc.VectorSubcoreMesh(core_axis_name="c", subcore_axis_name="s")

    @functools.partial(
        pl.kernel,
        mesh=mesh,
        out_type=jax.ShapeDtypeStruct((SEQ, EMB, BATCH), jnp.float32),
        scratch_types=[
            pltpu.VMEM((SEQ, SBLK), jnp.int32),
            [pltpu.VMEM((SBLK, PAD_W), jnp.float32) for _ in range(NG)],
            [pltpu.VMEM((EMB, SBLK), jnp.float32) for _ in range(NT)],
            [pltpu.VMEM((SBLK,), jnp.int32) for _ in range(NG)],
            [pltpu.SemaphoreType.DMA for _ in range(NG)],
            [pltpu.SemaphoreType.DMA for _ in range(NT)],
        ],
        compiler_params=pltpu.CompilerParams(
            use_tc_tiling_on_sc=True, needs_layout_passes=False),
    )
    def k(idx_hbm, table_hbm, out_hbm, idx_v, gbuf, tbuf, kidx, gsem, osem):
        wid = lax.axis_index("s") * NUM_CORES + lax.axis_index("c")
        s0 = wid * SBLK

        pltpu.sync_copy(idx_hbm.at[:, pl.ds(s0, SBLK)], idx_v)

        lanes = lax.iota(jnp.int32, 16)
        rot = [jax.lax.rem(lanes + jnp.int32(kk), jnp.int32(16))
               for kk in range(16)]

        def gather_desc(g):
            return pltpu.make_async_copy(
                table_hbm.at[kidx[g]], gbuf[g], gsem[g])

        def put_desc(t, b):
            return pltpu.make_async_copy(
                tbuf[b], out_hbm.at[t, :, pl.ds(s0, SBLK)], osem[b])

        def prep_gather(t, g):
            for m in range(SBLK // 16):
                iv = idx_v[t, pl.ds(m * 16, 16)]
                kidx[g][pl.ds(m * 16, 16)] = jax.lax.shift_right_logical(iv, 1)
            gather_desc(g).start()

        def transpose_block(t, g, b):
            def mbody(m, carry):
                off = m * 16
                iv = idx_v[t, pl.ds(off, 16)]
                pbase = jax.lax.shift_left(
                    jax.lax.bitwise_and(iv, jnp.int32(1)), 6)
                rows_m = off + lanes
                for db in range(EMB // 16):
                    d0 = db * 16
                    vs = [plsc.load_gather(
                        gbuf[g], [rows_m, pbase + jnp.int32(d0) + rot[kk]])
                        for kk in range(16)]
                    for kk in range(16):
                        plsc.store_scatter(
                            tbuf[b],
                            [jnp.int32(d0) + rot[kk], off + lanes],
                            vs[kk])
                return carry

            lax.fori_loop(0, SBLK // 16, mbody, 0)

        def step(t, gu, first=False, last=False):
            g = gu % NG
            b = gu % NT
            if not last:
                prep_gather(t + 2, (gu + 2) % NG)
            if not first:
                put_desc(t - NT, b).wait()
            gather_desc(g).wait()
            transpose_block(t, g, b)
            put_desc(t, b).start()

        prep_gather(0, 0)
        prep_gather(1, 1)
        for u in range(4):
            step(u, u, first=True)

        def body(r, carry):
            t = 4 + r * 4
            for u in range(4):
                step(t + u, u)
            return carry

        lax.fori_loop(0, (SEQ - 8) // 4, body, 0)

        for t in range(SEQ - 4, SEQ):
            step(t, t % 4, last=(t + 2 >= SEQ))
        for t in range(SEQ - 4, SEQ):
            put_desc(t, t % NT).wait()

    return k


def kernel(indices, weight):
    idx_t = jnp.transpose(indices)
    table = weight.reshape(VOCAB // 2, PAD_W)
    out_t = _make_gather()(idx_t, table)
    return jnp.transpose(out_t, (2, 0, 1))
